# Initial kernel scaffold; baseline (speedup 1.0000x reference)
import jax
import jax.numpy as jnp
from jax.experimental import pallas as pl


def kernel(x, edge_index, params, eps_noise):
    raise NotImplementedError("write your pallas kernel here")


# speedup vs baseline: 4.0053x; 4.0053x over previous
import functools

import jax
import jax.numpy as jnp
from jax import lax
from jax.experimental import pallas as pl
from jax.experimental.pallas import tpu as pltpu
from jax.experimental.pallas import tpu_sc as plsc

N = 10000
E = 320000
F = 128
EPS = 1e-10

NC = 2
NS = 16
NT = NC * NS
L = 16
EPT = E // NT
CE = 128
NCH = 80
PAD = NCH * CE - EPT
NROW = N + L
RPT = 624
RLAST = N - (NS - 1) * RPT

PREC = jax.lax.Precision.HIGHEST


@functools.lru_cache(maxsize=None)
def _make_agg(nch):
    mesh = plsc.VectorSubcoreMesh(core_axis_name="c", subcore_axis_name="s",
                                  num_cores=NC, num_subcores=NS)
    out_type = [jax.ShapeDtypeStruct((NC, N, F), jnp.float32) for _ in range(nch)]
    scratch = [
        pltpu.VMEM((NCH, CE), jnp.int32),
        pltpu.VMEM((CE,), jnp.int32),
        pltpu.VMEM((CE,), jnp.int32),
        pltpu.VMEM((CE, F), jnp.float32),
        pltpu.VMEM((CE, F), jnp.float32),
        pltpu.VMEM_SHARED((NROW, F), jnp.float32),
        pltpu.SemaphoreType.DMA,
        pltpu.SemaphoreType.DMA,
        pltpu.SemaphoreType.DMA,
        pltpu.SemaphoreType.DMA,
    ]

    @functools.partial(pl.kernel, out_type=tuple(out_type), mesh=mesh,
                       scratch_types=tuple(scratch),
                       compiler_params=pltpu.CompilerParams(
                           needs_layout_passes=False))
    def agg(*refs):
        ys = refs[:nch]
        k = nch
        src_hbm, dst_hbm, zeros_hbm = refs[k:k + 3]
        k += 3
        outs = refs[k:k + nch]
        k += nch
        (dstr, srcA, srcB, rowsA, rowsB, acc,
         semA, semB, semIA, semIB) = refs[k:k + 10]

        c = lax.axis_index("c")
        s = lax.axis_index("s")
        wid = c * NS + s
        base = wid * (NCH * CE)
        row0 = s * RPT
        last = s == NS - 1

        pltpu.sync_copy(dst_hbm.at[wid], dstr)

        def wait_sem(buf, sem):
            pltpu.make_async_copy(ys[0].at[pl.ds(0, CE)], buf, sem).wait()

        def wait_idx(buf, sem):
            pltpu.make_async_copy(src_hbm.at[pl.ds(0, CE)], buf, sem).wait()

        for cc in range(nch):
            out = outs[cc]
            y = ys[cc]

            @pl.when(last)
            def _():
                pltpu.sync_copy(zeros_hbm, acc.at[pl.ds(row0, RLAST)])

            @pl.when(jnp.logical_not(last))
            def _():
                pltpu.sync_copy(zeros_hbm.at[pl.ds(0, RPT)],
                                acc.at[pl.ds(row0, RPT)])
            plsc.subcore_barrier()

            pltpu.sync_copy(src_hbm.at[pl.ds(base, CE)], srcA)
            pltpu.async_copy(y.at[srcA], rowsA, semA)
            pltpu.sync_copy(src_hbm.at[pl.ds(base + CE, CE)], srcB)
            pltpu.async_copy(y.at[srcB], rowsB, semB)

            def pair(it, _):
                g = 2 * it
                wait_sem(rowsA, semA)
                pltpu.async_copy(src_hbm.at[pl.ds(base + (g + 2) * CE, CE)],
                                 srcA, semIA)
                pltpu.sync_copy(rowsA, acc.at[dstr.at[g]], add=True)
                wait_idx(srcA, semIA)
                pltpu.async_copy(y.at[srcA], rowsA, semA)
                wait_sem(rowsB, semB)
                pltpu.async_copy(src_hbm.at[pl.ds(base + (g + 3) * CE, CE)],
                                 srcB, semIB)
                pltpu.sync_copy(rowsB, acc.at[dstr.at[g + 1]], add=True)
                wait_idx(srcB, semIB)
                pltpu.async_copy(y.at[srcB], rowsB, semB)
                return _
            lax.fori_loop(0, NCH // 2 - 1, pair, None)

            wait_sem(rowsA, semA)
            pltpu.sync_copy(rowsA, acc.at[dstr.at[NCH - 2]], add=True)
            wait_sem(rowsB, semB)
            pltpu.sync_copy(rowsB, acc.at[dstr.at[NCH - 1]], add=True)

            plsc.subcore_barrier()

            @pl.when(last)
            def _():
                pltpu.sync_copy(acc.at[pl.ds(row0, RLAST)],
                                out.at[c, pl.ds(row0, RLAST)])

            @pl.when(jnp.logical_not(last))
            def _():
                pltpu.sync_copy(acc.at[pl.ds(row0, RPT)],
                                out.at[c, pl.ds(row0, RPT)])

    return agg


@functools.lru_cache(maxsize=None)
def _make_deg():
    mesh = plsc.VectorSubcoreMesh(core_axis_name="c", subcore_axis_name="s",
                                  num_cores=NC, num_subcores=NS)

    @functools.partial(
        pl.kernel,
        out_type=(jax.ShapeDtypeStruct((NT, NCH, CE), jnp.float32),),
        mesh=mesh,
        scratch_types=(pltpu.VMEM((NCH * CE,), jnp.int32),
                       pltpu.VMEM((NCH, CE), jnp.float32)),
        compiler_params=pltpu.CompilerParams(needs_layout_passes=False))
    def deg(dstf_hbm, deg_out, dstf, degt):
        c = lax.axis_index("c")
        s = lax.axis_index("s")
        wid = c * NS + s
        pltpu.sync_copy(dstf_hbm.at[pl.ds(wid * NCH * CE, NCH * CE)], dstf)
        zero16 = jnp.zeros((L,), jnp.float32)
        one16 = jnp.ones((L,), jnp.float32)

        def zbody(j, _):
            degt[j // 8, pl.ds((j % 8) * L, L)] = zero16
            return _
        lax.fori_loop(0, NCH * 8, zbody, None)

        def dbody(j, _):
            idx = dstf[pl.ds(j * L, L)]
            hi = lax.shift_right_logical(idx, 7)
            lo = jnp.bitwise_and(idx, CE - 1)
            plsc.addupdate_scatter(degt, [hi, lo], one16)
            return _
        lax.fori_loop(0, (NCH * CE) // L, dbody, None)
        pltpu.sync_copy(degt, deg_out.at[wid])

    return deg


R = 1000
GRID = N // R
_row = pl.BlockSpec((R, F), lambda i: (i, 0))
_row2 = pl.BlockSpec((NC, R, F), lambda i: (0, i, 0))
_full = lambda *shape: pl.BlockSpec(shape, lambda i: tuple(0 for _ in shape))
_col1 = pl.BlockSpec((R, 1), lambda i: (i, 0))


def _dot(a, b):
    return jnp.dot(a, b, precision=PREC, preferred_element_type=jnp.float32)


def _softplus(x):
    return jnp.maximum(x, 0.0) + jnp.log1p(jnp.exp(-jnp.abs(x)))


def _stage_a(x_r, hw_r, Wlin, blin, Q, r, We0, We1, benc, Wp, bp, Wpm, bpm,
             Wps, bps, xl_o, h_o, yenc_o, pm_o, ps_o):
    x = x_r[...]
    xl = _dot(x, Wlin[...]) + blin[...]
    hw = hw_r[...]
    e = [_dot(jnp.tanh(_dot(hw[kk], Q[...])), r[...]) for kk in range(3)]
    em = jnp.maximum(jnp.maximum(e[0], e[1]), e[2])
    ex = [jnp.exp(ei - em) for ei in e]
    se = 3.0 * (ex[0] + ex[1] + ex[2])
    h = (ex[0] * hw[0] + ex[1] * hw[1] + ex[2] * hw[2]) / se
    prior = jnp.maximum(_dot(h, Wp[...]) + bp[...], 0.0)
    xl_o[...] = xl
    h_o[...] = h
    yenc_o[...] = _dot(xl, We0[...]) + _dot(h, We1[...]) + benc[...]
    pm_o[...] = _dot(prior, Wpm[...]) + bpm[...]
    ps_o[...] = _softplus(_dot(prior, Wps[...]) + bps[...])


def _stage_deg(degp_r, dinv_o):
    deg = jnp.sum(degp_r[...], axis=0)
    dinv_o[...] = 1.0 / jnp.maximum(deg, 1.0)


def _stage_b(aggenc_r, dinv_r, Wmu, bmu, Wls, bls, ymu_o, ystd_o):
    aggenc = aggenc_r[0] + aggenc_r[1]
    dinv = dinv_r[...]
    enc = jnp.maximum(aggenc * dinv, 0.0)
    ymu_o[...] = _dot(enc, Wmu[...]) + bmu[...]
    ystd_o[...] = _dot(enc, Wls[...]) + bls[...]


def _stage_c(aggmu_r, aggls_r, dinv_r, eps_r, xl_r, h_r, Wlinz, blinz,
             Wx0, Wz1, Whc, bcat, pm_r, ps_r,
             yxz_o, yxr_o, yxh_o, kld_o):
    i = pl.program_id(0)
    dinv = dinv_r[...]
    mu = (aggmu_r[0] + aggmu_r[1]) * dinv
    std = _softplus((aggls_r[0] + aggls_r[1]) * dinv)
    z = eps_r[...] * std + mu
    zl = _dot(z, Wlinz[...]) + blinz[...]
    yall = _dot(xl_r[...], Wx0[...]) + _dot(zl, Wz1[...]) + bcat[...]
    yh = _dot(h_r[...], Whc[...])
    yxz_o[...] = yall[:, :F] + yh[:, :F]
    yxr_o[...] = yall[:, F:2 * F] + yh[:, F:]
    yxh_o[...] = yall[:, 2 * F:]
    pm = pm_r[...]
    ps = ps_r[...]
    kld_el = (2.0 * jnp.log(ps + EPS) - 2.0 * jnp.log(std + EPS)
              + ((std + EPS) ** 2 + (mu - pm) ** 2) / (ps + EPS) ** 2 - 1.0)
    part = jnp.full((1, 1), 0.5 / N / N) * jnp.sum(kld_el)

    @pl.when(i == 0)
    def _():
        kld_o[...] = jnp.zeros((1, 1), jnp.float32)
    kld_o[...] += part


def _stage_d(aggxz_r, aggxr_r, aggxh_r, dinv_r, h_r, Whh, bhh,
             zg_o, xhp_o, yhh_o):
    dinv = dinv_r[...]
    zg = jax.nn.sigmoid((aggxz_r[0] + aggxz_r[1]) * dinv)
    rg = jax.nn.sigmoid((aggxr_r[0] + aggxr_r[1]) * dinv)
    zg_o[...] = zg
    xhp_o[...] = (aggxh_r[0] + aggxh_r[1]) * dinv
    yhh_o[...] = _dot(rg * h_r[...], Whh[...]) + bhh[...]


def _stage_e(agghh_r, dinv_r, xhp_r, zg_r, h_r, hnew_o):
    ht = jnp.tanh(xhp_r[...] + (agghh_r[0] + agghh_r[1]) * dinv_r[...])
    zg = zg_r[...]
    hnew_o[...] = zg * h_r[...] + (1.0 - zg) * ht


def _nf32(*shape):
    return jax.ShapeDtypeStruct(shape, jnp.float32)


def kernel(x, edge_index, params, eps_noise):
    p = params
    src = edge_index[0].astype(jnp.int32)
    dst = edge_index[1].astype(jnp.int32)
    src_p = jnp.pad(src.reshape(NT, EPT), ((0, 0), (0, PAD))).reshape(-1)
    dst_p = jnp.pad(dst.reshape(NT, EPT), ((0, 0), (0, PAD)),
                    constant_values=N).reshape(NT, NCH, CE)
    dst_f = dst_p.reshape(-1)
    hw3 = p['hidden_window'].reshape(3, N, F)
    zrows = jnp.zeros((RLAST, F), jnp.float32)
    r1 = lambda b: b.reshape(1, F)

    xl, h, y_enc, pm, ps = pl.pallas_call(
        _stage_a,
        grid=(GRID,),
        in_specs=[_row, pl.BlockSpec((3, R, F), lambda i: (0, i, 0)),
                  _full(F, F), _full(1, F), _full(F, F), _full(F, 1),
                  _full(F, F), _full(F, F), _full(1, F),
                  _full(F, F), _full(1, F), _full(F, F), _full(1, F),
                  _full(F, F), _full(1, F)],
        out_specs=[_row, _row, _row, _row, _row],
        out_shape=[_nf32(N, F)] * 5,
    )(x, hw3, p['W_lin'], r1(p['b_lin']), p['Q'], p['r'],
      p['W_enc'][:F], p['W_enc'][F:], r1(p['b_enc']),
      p['W_p'], r1(p['b_p']), p['W_pm'], r1(p['b_pm']),
      p['W_ps'], r1(p['b_ps']))

    (deg_parts,) = _make_deg()(dst_f)
    (agg_enc,) = _make_agg(1)(y_enc, src_p, dst_p, zrows)

    dinv_grid = pl.pallas_call(
        _stage_deg,
        grid=(1,),
        in_specs=[_full(NT, NCH, CE)],
        out_specs=_full(NCH, CE),
        out_shape=_nf32(NCH, CE),
    )(deg_parts)
    dinv = dinv_grid.reshape(NCH * CE, 1)[:N]

    y_mu, y_std = pl.pallas_call(
        _stage_b,
        grid=(GRID,),
        in_specs=[_row2, _col1,
                  _full(F, F), _full(1, F), _full(F, F), _full(1, F)],
        out_specs=[_row, _row],
        out_shape=[_nf32(N, F), _nf32(N, F)],
    )(agg_enc, dinv, p['W_mu'], r1(p['b_mu']), p['W_ls'], r1(p['b_ls']))

    agg_mu, agg_ls = _make_agg(2)(y_mu, y_std, src_p, dst_p, zrows)

    Wx0 = jnp.concatenate([p['W_xz'][:F], p['W_xr'][:F], p['W_xh'][:F]], axis=1)
    Wz1 = jnp.concatenate([p['W_xz'][F:], p['W_xr'][F:], p['W_xh'][F:]], axis=1)
    Whc = jnp.concatenate([p['W_hz'], p['W_hr']], axis=1)
    bcat = jnp.concatenate([p['b_xz'] + p['b_hz'], p['b_xr'] + p['b_hr'],
                            p['b_xh']]).reshape(1, 3 * F)

    y_xz, y_xr, y_xh, kld = pl.pallas_call(
        _stage_c,
        grid=(GRID,),
        in_specs=[_row2, _row2, _col1, _row, _row, _row,
                  _full(F, F), _full(1, F),
                  _full(F, 3 * F), _full(F, 3 * F), _full(F, 2 * F),
                  _full(1, 3 * F), _row, _row],
        out_specs=[_row, _row, _row, _full(1, 1)],
        out_shape=[_nf32(N, F), _nf32(N, F), _nf32(N, F), _nf32(1, 1)],
    )(agg_mu, agg_ls, dinv, eps_noise, xl, h, p['W_linz'], r1(p['b_linz']),
      Wx0, Wz1, Whc, bcat, pm, ps)

    agg_xz, agg_xr, agg_xh = _make_agg(3)(y_xz, y_xr, y_xh,
                                          src_p, dst_p, zrows)

    zg, xh_part, y_hh = pl.pallas_call(
        _stage_d,
        grid=(GRID,),
        in_specs=[_row2, _row2, _row2, _col1, _row, _full(F, F), _full(1, F)],
        out_specs=[_row, _row, _row],
        out_shape=[_nf32(N, F)] * 3,
    )(agg_xz, agg_xr, agg_xh, dinv, h, p['W_hh'], r1(p['b_hh']))

    (agg_hh,) = _make_agg(1)(y_hh, src_p, dst_p, zrows)

    h_new = pl.pallas_call(
        _stage_e,
        grid=(GRID,),
        in_specs=[_row2, _col1, _row, _row, _row],
        out_specs=_row,
        out_shape=_nf32(N, F),
    )(agg_hh, dinv, xh_part, zg, h)

    return h_new, kld.reshape(())


# speedup vs baseline: 4.7395x; 1.1833x over previous
import functools

import jax
import jax.numpy as jnp
from jax import lax
from jax.experimental import pallas as pl
from jax.experimental.pallas import tpu as pltpu
from jax.experimental.pallas import tpu_sc as plsc

N = 10000
E = 320000
F = 128
EPS = 1e-10

NC = 2
NS = 16
NT = NC * NS
L = 16
EPT = E // NT
CE = 128
NCH = 80
PAD = NCH * CE - EPT
NROW = N + L
RPT = 624
RLAST = N - (NS - 1) * RPT

PREC = jax.lax.Precision.HIGHEST


@functools.lru_cache(maxsize=None)
def _make_agg(nch):
    mesh = plsc.VectorSubcoreMesh(core_axis_name="c", subcore_axis_name="s",
                                  num_cores=NC, num_subcores=NS)
    out_type = [jax.ShapeDtypeStruct((NC, N, F), jnp.float32) for _ in range(nch)]
    scratch = (
        pltpu.VMEM((NCH, CE), jnp.int32),
        pltpu.VMEM((NCH, CE), jnp.int32),
        pltpu.VMEM((CE, F), jnp.float32),
        pltpu.VMEM_SHARED((NROW, F), jnp.float32),
        pltpu.SemaphoreType.DMA,
    )

    @functools.partial(pl.kernel, out_type=tuple(out_type), mesh=mesh,
                       scratch_types=scratch,
                       compiler_params=pltpu.CompilerParams(
                           needs_layout_passes=False))
    def agg(*refs):
        ys = refs[:nch]
        k = nch
        src_hbm, dst_hbm, zeros_hbm = refs[k:k + 3]
        k += 3
        outs = refs[k:k + nch]
        k += nch
        (srcr, dstr, rows, acc, sem) = refs[k:k + 5]

        c = lax.axis_index("c")
        s = lax.axis_index("s")
        wid = c * NS + s
        row0 = s * RPT
        last = s == NS - 1

        pltpu.sync_copy(src_hbm.at[wid], srcr)
        pltpu.sync_copy(dst_hbm.at[wid], dstr)

        for cc in range(nch):
            out = outs[cc]
            y = ys[cc]

            @pl.when(last)
            def _():
                pltpu.sync_copy(zeros_hbm, acc.at[pl.ds(row0, RLAST)])

            @pl.when(jnp.logical_not(last))
            def _():
                pltpu.sync_copy(zeros_hbm.at[pl.ds(0, RPT)],
                                acc.at[pl.ds(row0, RPT)])
            plsc.subcore_barrier()

            def echunk(g, _):
                pltpu.async_copy(y.at[srcr.at[g]], rows, sem).wait()
                pltpu.sync_copy(rows, acc.at[dstr.at[g]], add=True)
                return _
            lax.fori_loop(0, NCH, echunk, None)

            plsc.subcore_barrier()

            @pl.when(last)
            def _():
                pltpu.sync_copy(acc.at[pl.ds(row0, RLAST)],
                                out.at[c, pl.ds(row0, RLAST)])

            @pl.when(jnp.logical_not(last))
            def _():
                pltpu.sync_copy(acc.at[pl.ds(row0, RPT)],
                                out.at[c, pl.ds(row0, RPT)])

    return agg


@functools.lru_cache(maxsize=None)
def _make_deg():
    mesh = plsc.VectorSubcoreMesh(core_axis_name="c", subcore_axis_name="s",
                                  num_cores=NC, num_subcores=NS)

    @functools.partial(
        pl.kernel,
        out_type=(jax.ShapeDtypeStruct((NT, NCH, CE), jnp.float32),),
        mesh=mesh,
        scratch_types=(pltpu.VMEM((NCH * CE,), jnp.int32),
                       pltpu.VMEM((NCH, CE), jnp.float32)),
        compiler_params=pltpu.CompilerParams(needs_layout_passes=False))
    def deg(dstf_hbm, deg_out, dstf, degt):
        c = lax.axis_index("c")
        s = lax.axis_index("s")
        wid = c * NS + s
        pltpu.sync_copy(dstf_hbm.at[pl.ds(wid * NCH * CE, NCH * CE)], dstf)
        zero16 = jnp.zeros((L,), jnp.float32)
        one16 = jnp.ones((L,), jnp.float32)

        def zbody(j, _):
            degt[j // 8, pl.ds((j % 8) * L, L)] = zero16
            return _
        lax.fori_loop(0, NCH * 8, zbody, None)

        def dbody(j, _):
            idx = dstf[pl.ds(j * L, L)]
            hi = lax.shift_right_logical(idx, 7)
            lo = jnp.bitwise_and(idx, CE - 1)
            plsc.addupdate_scatter(degt, [hi, lo], one16)
            return _
        lax.fori_loop(0, (NCH * CE) // L, dbody, None)
        pltpu.sync_copy(degt, deg_out.at[wid])

    return deg


R = 1000
GRID = N // R
_row = pl.BlockSpec((R, F), lambda i: (i, 0))
_row2 = pl.BlockSpec((NC, R, F), lambda i: (0, i, 0))
_full = lambda *shape: pl.BlockSpec(shape, lambda i: tuple(0 for _ in shape))
_col1 = pl.BlockSpec((R, 1), lambda i: (i, 0))


def _dot(a, b):
    return jnp.dot(a, b, precision=PREC, preferred_element_type=jnp.float32)


def _softplus(x):
    return jnp.maximum(x, 0.0) + jnp.log1p(jnp.exp(-jnp.abs(x)))


def _stage_a(x_r, hw_r, Wlin, blin, Q, r, Wp, bp, Wpm, bpm, Wps, bps,
             xl_o, h_o, pm_o, ps_o):
    x = x_r[...]
    xl = _dot(x, Wlin[...]) + blin[...]
    hw = hw_r[...]
    e = [_dot(jnp.tanh(_dot(hw[kk], Q[...])), r[...]) for kk in range(3)]
    em = jnp.maximum(jnp.maximum(e[0], e[1]), e[2])
    ex = [jnp.exp(ei - em) for ei in e]
    se = 3.0 * (ex[0] + ex[1] + ex[2])
    h = (ex[0] * hw[0] + ex[1] * hw[1] + ex[2] * hw[2]) / se
    prior = jnp.maximum(_dot(h, Wp[...]) + bp[...], 0.0)
    xl_o[...] = xl
    h_o[...] = h
    pm_o[...] = _dot(prior, Wpm[...]) + bpm[...]
    ps_o[...] = _softplus(_dot(prior, Wps[...]) + bps[...])


def _stage_deg(degp_r, dinv_o, bm_o):
    deg = jnp.sum(degp_r[...], axis=0)
    dinv_o[...] = 1.0 / jnp.maximum(deg, 1.0)
    bm_o[...] = (deg > 0.0).astype(jnp.float32)


def _stage_b(axl_r, ah_r, dinv_r, bm_r, We0, We1, benc, yenc_o):
    axl = axl_r[0] + axl_r[1]
    ah = ah_r[0] + ah_r[1]
    pre = (_dot(axl, We0[...]) + _dot(ah, We1[...])) * dinv_r[...]
    yenc_o[...] = jnp.maximum(pre + bm_r[...] * benc[...], 0.0)


def _stage_c(aenc_r, dinv_r, bm_r, eps_r, Wmu, bmu, Wls, bls, Wlinz, blinz,
             pm_r, ps_r, zl_o, kld_o):
    i = pl.program_id(0)
    aenc = (aenc_r[0] + aenc_r[1]) * dinv_r[...]
    bm = bm_r[...]
    mu = _dot(aenc, Wmu[...]) + bm * bmu[...]
    std = _softplus(_dot(aenc, Wls[...]) + bm * bls[...])
    z = eps_r[...] * std + mu
    zl_o[...] = _dot(z, Wlinz[...]) + blinz[...]
    pm = pm_r[...]
    ps = ps_r[...]
    kld_el = (2.0 * jnp.log(ps + EPS) - 2.0 * jnp.log(std + EPS)
              + ((std + EPS) ** 2 + (mu - pm) ** 2) / (ps + EPS) ** 2 - 1.0)
    part = jnp.full((1, 1), 0.5 / N / N) * jnp.sum(kld_el)

    @pl.when(i == 0)
    def _():
        kld_o[...] = jnp.zeros((1, 1), jnp.float32)
    kld_o[...] += part


def _stage_d(axl_r, azl_r, ah_r, dinv_r, bm_r, h_r, Wx0, Wz1, Whc, bcat,
             zg_o, xhp_o, y4_o):
    dinv = dinv_r[...]
    bm = bm_r[...]
    yall = _dot(axl_r[0] + axl_r[1], Wx0[...]) \
        + _dot(azl_r[0] + azl_r[1], Wz1[...])
    yh = _dot(ah_r[0] + ah_r[1], Whc[...])
    pre_z = (yall[:, :F] + yh[:, :F]) * dinv + bm * bcat[:, :F]
    pre_r = (yall[:, F:2 * F] + yh[:, F:]) * dinv + bm * bcat[:, F:2 * F]
    zg = jax.nn.sigmoid(pre_z)
    rg = jax.nn.sigmoid(pre_r)
    zg_o[...] = zg
    xhp_o[...] = yall[:, 2 * F:] * dinv + bm * bcat[:, 2 * F:]
    y4_o[...] = rg * h_r[...]


def _stage_e(a4_r, dinv_r, bm_r, Whh, bhh, xhp_r, zg_r, h_r, hnew_o):
    pre = _dot(a4_r[0] + a4_r[1], Whh[...]) * dinv_r[...] \
        + bm_r[...] * bhh[...]
    ht = jnp.tanh(xhp_r[...] + pre)
    zg = zg_r[...]
    hnew_o[...] = zg * h_r[...] + (1.0 - zg) * ht


def _nf32(*shape):
    return jax.ShapeDtypeStruct(shape, jnp.float32)


def kernel(x, edge_index, params, eps_noise):
    p = params
    src = edge_index[0].astype(jnp.int32)
    dst = edge_index[1].astype(jnp.int32)
    src_p = jnp.pad(src.reshape(NT, EPT), ((0, 0), (0, PAD))
                    ).reshape(NT, NCH, CE)
    dst_p = jnp.pad(dst.reshape(NT, EPT), ((0, 0), (0, PAD)),
                    constant_values=N).reshape(NT, NCH, CE)
    dst_f = dst_p.reshape(-1)
    hw3 = p['hidden_window'].reshape(3, N, F)
    zrows = jnp.zeros((RLAST, F), jnp.float32)
    r1 = lambda b: b.reshape(1, F)

    (deg_parts,) = _make_deg()(dst_f)

    xl, h, pm, ps = pl.pallas_call(
        _stage_a,
        grid=(GRID,),
        in_specs=[_row, pl.BlockSpec((3, R, F), lambda i: (0, i, 0)),
                  _full(F, F), _full(1, F), _full(F, F), _full(F, 1),
                  _full(F, F), _full(1, F), _full(F, F), _full(1, F),
                  _full(F, F), _full(1, F)],
        out_specs=[_row, _row, _row, _row],
        out_shape=[_nf32(N, F)] * 4,
    )(x, hw3, p['W_lin'], r1(p['b_lin']), p['Q'], p['r'],
      p['W_p'], r1(p['b_p']), p['W_pm'], r1(p['b_pm']),
      p['W_ps'], r1(p['b_ps']))

    a_xl, a_h = _make_agg(2)(xl, h, src_p, dst_p, zrows)

    dinv_grid, bm_grid = pl.pallas_call(
        _stage_deg,
        grid=(1,),
        in_specs=[_full(NT, NCH, CE)],
        out_specs=[_full(NCH, CE), _full(NCH, CE)],
        out_shape=[_nf32(NCH, CE), _nf32(NCH, CE)],
    )(deg_parts)
    dinv = dinv_grid.reshape(NCH * CE, 1)[:N]
    bm = bm_grid.reshape(NCH * CE, 1)[:N]

    enc = pl.pallas_call(
        _stage_b,
        grid=(GRID,),
        in_specs=[_row2, _row2, _col1, _col1,
                  _full(F, F), _full(F, F), _full(1, F)],
        out_specs=_row,
        out_shape=_nf32(N, F),
    )(a_xl, a_h, dinv, bm, p['W_enc'][:F], p['W_enc'][F:], r1(p['b_enc']))

    (a_enc,) = _make_agg(1)(enc, src_p, dst_p, zrows)

    zl, kld = pl.pallas_call(
        _stage_c,
        grid=(GRID,),
        in_specs=[_row2, _col1, _col1, _row,
                  _full(F, F), _full(1, F), _full(F, F), _full(1, F),
                  _full(F, F), _full(1, F), _row, _row],
        out_specs=[_row, _full(1, 1)],
        out_shape=[_nf32(N, F), _nf32(1, 1)],
    )(a_enc, dinv, bm, eps_noise, p['W_mu'], r1(p['b_mu']),
      p['W_ls'], r1(p['b_ls']), p['W_linz'], r1(p['b_linz']), pm, ps)

    (a_zl,) = _make_agg(1)(zl, src_p, dst_p, zrows)

    Wx0 = jnp.concatenate([p['W_xz'][:F], p['W_xr'][:F], p['W_xh'][:F]], axis=1)
    Wz1 = jnp.concatenate([p['W_xz'][F:], p['W_xr'][F:], p['W_xh'][F:]], axis=1)
    Whc = jnp.concatenate([p['W_hz'], p['W_hr']], axis=1)
    bcat = jnp.concatenate([p['b_xz'] + p['b_hz'], p['b_xr'] + p['b_hr'],
                            p['b_xh']]).reshape(1, 3 * F)

    zg, xh_part, y4 = pl.pallas_call(
        _stage_d,
        grid=(GRID,),
        in_specs=[_row2, _row2, _row2, _col1, _col1, _row,
                  _full(F, 3 * F), _full(F, 3 * F), _full(F, 2 * F),
                  _full(1, 3 * F)],
        out_specs=[_row, _row, _row],
        out_shape=[_nf32(N, F)] * 3,
    )(a_xl, a_zl, a_h, dinv, bm, h, Wx0, Wz1, Whc, bcat)

    (a_4,) = _make_agg(1)(y4, src_p, dst_p, zrows)

    h_new = pl.pallas_call(
        _stage_e,
        grid=(GRID,),
        in_specs=[_row2, _col1, _col1, _full(F, F), _full(1, F),
                  _row, _row, _row],
        out_specs=_row,
        out_shape=_nf32(N, F),
    )(a_4, dinv, bm, p['W_hh'], r1(p['b_hh']), xh_part, zg, h)

    return h_new, kld.reshape(())
